# baseline (speedup 1.0000x reference)
import functools

import jax
import jax.numpy as jnp
from jax import lax
from jax.experimental import pallas as pl
from jax.experimental.pallas import tpu as pltpu
from jax.experimental.pallas import tpu_sc as plsc

N = 10000
NP = 10240
E = 320000
F = 128
T = 12
OUT = 32

NC, NS, L = 2, 16, 16
NW = NC * NS
EPT = E // NW
CK = 128
NCHUNK = (2 * OUT * T) // CK
B = 40
NBATCH = EPT // B
SB = 50
NSB = NBATCH // SB
NPAIR = SB // 2
SLICE = NP // NS
ZROWS = 128

_f32 = jnp.float32
_i32 = jnp.int32

_SC_MESH = plsc.VectorSubcoreMesh(
    core_axis_name="c", subcore_axis_name="s", num_cores=NC, num_subcores=NS)
_SC_PARAMS = pltpu.CompilerParams(needs_layout_passes=False)


_EROWS = 1600
_EBLK = 8 * _EROWS


def _ew_body(ef_ref, row_ref, wt_ref, b_ref, out_ref, row6_ref):
    v = ef_ref[...] * wt_ref[...]
    s = jnp.sum(v.reshape(_EROWS, 8, 16), axis=2)
    out_ref[...] = jnp.maximum(s + b_ref[0, 0], 0.0)
    r = row_ref[0]
    for c in range(NCHUNK):
        row6_ref[c, 0] = r + c * NP


def _ew_call(ef, row, w, b):
    ef8 = ef.reshape(E // 8, 128)
    row3d = row.reshape(E // _EBLK, _EBLK // 128, 128)
    wt = jnp.tile(w[:, 0], 8).reshape(1, 128)
    ew2, row6 = pl.pallas_call(
        _ew_body,
        grid=(E // _EBLK,),
        in_specs=[
            pl.BlockSpec((_EROWS, 128), lambda i: (i, 0)),
            pl.BlockSpec((1, _EBLK // 128, 128), lambda i: (i, 0, 0)),
            pl.BlockSpec((1, 128), lambda i: (0, 0)),
            pl.BlockSpec((1, 1), lambda i: (0, 0)),
        ],
        out_specs=[
            pl.BlockSpec((_EROWS, 8), lambda i: (i, 0)),
            pl.BlockSpec((NCHUNK, 1, _EBLK // 128, 128),
                         lambda i: (0, i, 0, 0)),
        ],
        out_shape=[
            jax.ShapeDtypeStruct((E // 8, 8), _f32),
            jax.ShapeDtypeStruct((NCHUNK, E // _EBLK, _EBLK // 128, 128),
                                 _i32),
        ],
    )(ef8, row3d, wt, b)
    return ew2.reshape(E), row6.reshape(NCHUNK, NW, NSB, SB, B)


_DCH = 2000


def _deg_body(col_hbm, ew_hbm, out_hbm, col_v, ew_v, acc_v):
    cc = lax.axis_index("c")
    ss = lax.axis_index("s")
    tid = ss * NC + cc
    base = tid * EPT

    def zero(i, _):
        acc_v[pl.ds(i * L, L)] = jnp.zeros((L,), _f32)
        return 0

    lax.fori_loop(0, NP // L, zero, 0)

    def piece(q, _):
        pltpu.sync_copy(col_hbm.at[pl.ds(base + q * _DCH, _DCH)], col_v)
        pltpu.sync_copy(ew_hbm.at[pl.ds(base + q * _DCH, _DCH)], ew_v)

        def body(i, _):
            idx = col_v[pl.ds(i * L, L)]
            w = ew_v[pl.ds(i * L, L)]
            plsc.addupdate_scatter(acc_v, [idx], w)
            return 0

        lax.fori_loop(0, _DCH // L, body, 0)
        return 0

    lax.fori_loop(0, EPT // _DCH, piece, 0)
    pltpu.sync_copy(acc_v, out_hbm.at[tid])


_deg_call = functools.partial(
    pl.kernel,
    out_type=jax.ShapeDtypeStruct((NW, NP), _f32),
    mesh=_SC_MESH,
    compiler_params=_SC_PARAMS,
    scratch_types=[
        pltpu.VMEM((_DCH,), _i32),
        pltpu.VMEM((_DCH,), _f32),
        pltpu.VMEM((NP,), _f32),
    ],
)(_deg_body)


_NB = 2048


def _prep_body(xt_ref, degp_ref, wf_ref, wzh_ref, bf_ref, yp_ref, dis_ref):
    deg = jnp.sum(degp_ref[...], axis=0) + 1.0
    dis = lax.rsqrt(deg)
    dis_ref[...] = dis[None, :]
    wc = jnp.dot(wf_ref[...], wzh_ref[...], preferred_element_type=_f32)
    bc = jnp.dot(bf_ref[...], wzh_ref[...], preferred_element_type=_f32)
    for t in range(T):
        y = jnp.dot(xt_ref[t], wc, preferred_element_type=_f32) + bc
        o = (t % 2) * 64
        yp_ref[t // 2, :, o:o + 64] = y * dis[:, None]


def _prep_call(xt, degp, wf, wzh, bf):
    return pl.pallas_call(
        _prep_body,
        grid=(NP // _NB,),
        in_specs=[
            pl.BlockSpec((T, _NB, F), lambda i: (0, i, 0)),
            pl.BlockSpec((NW, _NB), lambda i: (0, i)),
            pl.BlockSpec((F, F), lambda i: (0, 0)),
            pl.BlockSpec((F, 2 * OUT), lambda i: (0, 0)),
            pl.BlockSpec((1, F), lambda i: (0, 0)),
        ],
        out_specs=[
            pl.BlockSpec((NCHUNK, _NB, CK), lambda i: (0, i, 0)),
            pl.BlockSpec((1, _NB), lambda i: (0, i)),
        ],
        out_shape=[
            jax.ShapeDtypeStruct((NCHUNK, NP, CK), _f32),
            jax.ShapeDtypeStruct((1, NP), _f32),
        ],
    )(xt, degp, wf, wzh, bf)


def _agg_body(ypf_hbm, row6_hbm, col3_hbm, ew3_hbm, out_hbm,
              colsb, rowsb, ewsb, gbuf0, gbuf1, sbuf, a_sh,
              sem_g0, sem_g1, sem_s):
    cc = lax.axis_index("c")
    ss = lax.axis_index("s")
    tid = ss * NC + cc
    gbufs = (gbuf0, gbuf1)
    gsems = (sem_g0, sem_g1)

    def gzero():
        def zrow(i, _):
            for k in range(CK // L):
                gbuf0[i, pl.ds(k * L, L)] = jnp.zeros((L,), _f32)
            return 0

        lax.fori_loop(0, B, zrow, 0)

    def zero_slice():
        for j in range(SLICE // B):
            pltpu.sync_copy(gbuf0, a_sh.at[pl.ds(ss * SLICE + j * B, B)])

    def scale(kb, gbuf):
        for e in range(B):
            w = plsc.load_gather(
                ewsb, [jnp.full((L,), kb, _i32), jnp.full((L,), e, _i32)])
            for k in range(CK // L):
                sl = pl.ds(k * L, L)
                sbuf[e, sl] = gbuf[e, sl] * w

    def wait_gather(kb, p):
        pltpu.make_async_copy(
            ypf_hbm.at[rowsb.at[kb]], gbufs[p], gsems[p]).wait()

    def issue_gather(kb, p):
        pltpu.async_copy(ypf_hbm.at[rowsb.at[kb]], gbufs[p], gsems[p])

    def issue_scatter(kb):
        pltpu.async_copy(sbuf, a_sh.at[colsb.at[kb]], sem_s, add=True)

    def drain_scatter():
        pltpu.make_async_copy(sbuf, a_sh.at[colsb.at[0]], sem_s).wait()

    gzero()
    zero_slice()
    plsc.subcore_barrier()

    def chunk(c, _):
        def superb(s, _):
            pltpu.sync_copy(row6_hbm.at[c, tid, s], rowsb)
            pltpu.sync_copy(col3_hbm.at[tid, s], colsb)
            pltpu.sync_copy(ew3_hbm.at[tid, s], ewsb)
            issue_gather(0, 0)
            issue_gather(1, 1)
            wait_gather(0, 0)
            scale(0, gbuf0)
            issue_scatter(0)
            issue_gather(2, 0)
            wait_gather(1, 1)
            drain_scatter()
            scale(1, gbuf1)
            issue_scatter(1)
            issue_gather(3, 1)

            def pair(jp, _):
                for p in range(2):
                    kb = 2 * jp + p
                    wait_gather(kb, p)
                    drain_scatter()
                    scale(kb, gbufs[p])
                    issue_scatter(kb)
                    issue_gather(kb + 2, p)
                return 0

            lax.fori_loop(1, NPAIR - 1, pair, 0)
            for p in range(2):
                kb = SB - 2 + p
                wait_gather(kb, p)
                drain_scatter()
                scale(kb, gbufs[p])
                issue_scatter(kb)
            drain_scatter()
            return 0

        lax.fori_loop(0, NSB, superb, 0)
        plsc.subcore_barrier()
        pltpu.sync_copy(a_sh.at[pl.ds(ss * SLICE, SLICE)],
                        out_hbm.at[cc, c, pl.ds(ss * SLICE, SLICE)])
        gzero()
        zero_slice()
        plsc.subcore_barrier()
        return 0

    lax.fori_loop(0, NCHUNK, chunk, 0)


_agg_call = functools.partial(
    pl.kernel,
    out_type=jax.ShapeDtypeStruct((NC, NCHUNK, NP, CK), _f32),
    mesh=_SC_MESH,
    compiler_params=_SC_PARAMS,
    scratch_types=[
        pltpu.VMEM((SB, B), _i32),
        pltpu.VMEM((SB, B), _i32),
        pltpu.VMEM((SB, B), _f32),
        pltpu.VMEM((B, CK), _f32),
        pltpu.VMEM((B, CK), _f32),
        pltpu.VMEM((B, CK), _f32),
        pltpu.VMEM_SHARED((NP, CK), _f32),
        pltpu.SemaphoreType.DMA,
        pltpu.SemaphoreType.DMA,
        pltpu.SemaphoreType.DMA,
    ],
)(_agg_body)


_NB2 = 2048


def _fin_body(p_ref, yp_ref, dis_ref, att_ref, wlz_ref, blz_ref,
              wlh_ref, blh_ref, bz_ref, bh_ref, wo_ref, bo_ref, out_ref):
    att = att_ref[0]
    ex = jnp.exp(att - jnp.max(att))
    probs = ex / jnp.sum(ex)
    dis = dis_ref[0]
    h = jnp.zeros((_NB2, OUT), _f32)
    parr = p_ref[...]
    for c in range(NCHUNK):
        ac = (parr[0, c] + parr[1, c] + yp_ref[c]) * dis[:, None]
        for tt in range(2):
            t = 2 * c + tt
            o = tt * 64
            uz = ac[:, o:o + OUT] + bz_ref[...]
            uh = ac[:, o + OUT:o + 2 * OUT] + bh_ref[...]
            z = jax.nn.sigmoid(
                jnp.dot(uz, wlz_ref[...], preferred_element_type=_f32)
                + blz_ref[...])
            ht = jnp.tanh(
                jnp.dot(uh, wlh_ref[...], preferred_element_type=_f32)
                + blh_ref[...])
            h = h + probs[t] * (1.0 - z) * ht
    out_ref[...] = (
        jnp.dot(jnp.maximum(h, 0.0), wo_ref[...], preferred_element_type=_f32)
        + bo_ref[...])


def _final_call(p, yp, dis, att, wlz, blz, wlh, blh, bz, bh, wo, bo):
    small = lambda r, ccol: pl.BlockSpec((r, ccol), lambda i: (0, 0))
    return pl.pallas_call(
        _fin_body,
        grid=(NP // _NB2,),
        in_specs=[
            pl.BlockSpec((NC, NCHUNK, _NB2, CK), lambda i: (0, 0, i, 0)),
            pl.BlockSpec((NCHUNK, _NB2, CK), lambda i: (0, i, 0)),
            pl.BlockSpec((1, _NB2), lambda i: (0, i)),
            small(1, T),
            small(OUT, OUT), small(1, OUT),
            small(OUT, OUT), small(1, OUT),
            small(1, OUT), small(1, OUT),
            small(OUT, T), small(1, T),
        ],
        out_specs=pl.BlockSpec((_NB2, T), lambda i: (i, 0)),
        out_shape=jax.ShapeDtypeStruct((NP, T), _f32),
    )(p, yp, dis, att, wlz, blz, wlh, blh, bz, bh, wo, bo)


def kernel(x, edge_index, edge_features, W_feat, b_feat, W_edge, b_edge,
           att, Wz, bz, Wlz, blz, Wr, br, Wlr, blr, Wh, bh, Wlh, blh,
           W_out, b_out):
    row = edge_index[0]
    col = edge_index[1]
    ew, row6 = _ew_call(edge_features, row, W_edge, b_edge.reshape(1, 1))
    degp = _deg_call(col, ew)
    xt = jnp.pad(jnp.transpose(x, (2, 0, 1)), ((0, 0), (0, NP - N), (0, 0)))
    wzh = jnp.concatenate([Wz, Wh], axis=1)
    yp, dis = _prep_call(xt, degp, W_feat, wzh, b_feat.reshape(1, F))
    col3 = col.reshape(NW, NSB, SB, B)
    ew3 = ew.reshape(NW, NSB, SB, B)
    p = _agg_call(yp.reshape(NCHUNK * NP, CK), row6, col3, ew3)
    out = _final_call(
        p, yp, dis, att.reshape(1, T),
        Wlz[:OUT], blz.reshape(1, OUT),
        Wlh[:OUT], blh.reshape(1, OUT),
        bz.reshape(1, OUT), bh.reshape(1, OUT),
        W_out, b_out.reshape(1, T))
    return out[:N]


# speedup vs baseline: 1.0355x; 1.0355x over previous
import functools

import jax
import jax.numpy as jnp
from jax import lax
from jax.experimental import pallas as pl
from jax.experimental.pallas import tpu as pltpu
from jax.experimental.pallas import tpu_sc as plsc

N = 10000
NP = 10240
E = 320000
F = 128
T = 12
OUT = 32

NC, NS, L = 2, 16, 16
NW = NC * NS
EPT = E // NW
CK = 128
NCHUNK = (2 * OUT * T) // CK
B = 40
NBATCH = EPT // B
SB = 50
NSB = NBATCH // SB
NPAIR = SB // 2
SLICE = NP // NS
ZROWS = 128

_f32 = jnp.float32
_i32 = jnp.int32

_SC_MESH = plsc.VectorSubcoreMesh(
    core_axis_name="c", subcore_axis_name="s", num_cores=NC, num_subcores=NS)
_SC_PARAMS = pltpu.CompilerParams(needs_layout_passes=False)


_EROWS = 1600
_EBLK = 8 * _EROWS


def _ew_body(ef_ref, row_ref, wt_ref, b_ref, out_ref, row6_ref):
    v = ef_ref[...] * wt_ref[...]
    s = jnp.sum(v.reshape(_EROWS, 8, 16), axis=2)
    out_ref[...] = jnp.maximum(s + b_ref[0, 0], 0.0)
    r = row_ref[0]
    for c in range(NCHUNK):
        row6_ref[c, 0] = r + c * NP


def _ew_call(ef, row, w, b):
    ef8 = ef.reshape(E // 8, 128)
    row3d = row.reshape(E // _EBLK, _EBLK // 128, 128)
    wt = jnp.tile(w[:, 0], 8).reshape(1, 128)
    ew2, row6 = pl.pallas_call(
        _ew_body,
        grid=(E // _EBLK,),
        in_specs=[
            pl.BlockSpec((_EROWS, 128), lambda i: (i, 0)),
            pl.BlockSpec((1, _EBLK // 128, 128), lambda i: (i, 0, 0)),
            pl.BlockSpec((1, 128), lambda i: (0, 0)),
            pl.BlockSpec((1, 1), lambda i: (0, 0)),
        ],
        out_specs=[
            pl.BlockSpec((_EROWS, 8), lambda i: (i, 0)),
            pl.BlockSpec((NCHUNK, 1, _EBLK // 128, 128),
                         lambda i: (0, i, 0, 0)),
        ],
        out_shape=[
            jax.ShapeDtypeStruct((E // 8, 8), _f32),
            jax.ShapeDtypeStruct((NCHUNK, E // _EBLK, _EBLK // 128, 128),
                                 _i32),
        ],
    )(ef8, row3d, wt, b)
    return ew2.reshape(E), row6.reshape(NCHUNK, NW, NSB, SB, B)


_DCH = 2000


def _deg_body(col_hbm, ew_hbm, out_hbm, col_v, ew_v, acc_v):
    cc = lax.axis_index("c")
    ss = lax.axis_index("s")
    tid = ss * NC + cc
    base = tid * EPT

    def zero(i, _):
        acc_v[pl.ds(i * L, L)] = jnp.zeros((L,), _f32)
        return 0

    lax.fori_loop(0, NP // L, zero, 0)

    def piece(q, _):
        pltpu.sync_copy(col_hbm.at[pl.ds(base + q * _DCH, _DCH)], col_v)
        pltpu.sync_copy(ew_hbm.at[pl.ds(base + q * _DCH, _DCH)], ew_v)

        def body(i, _):
            idx = col_v[pl.ds(i * L, L)]
            w = ew_v[pl.ds(i * L, L)]
            plsc.addupdate_scatter(acc_v, [idx], w)
            return 0

        lax.fori_loop(0, _DCH // L, body, 0)
        return 0

    lax.fori_loop(0, EPT // _DCH, piece, 0)
    pltpu.sync_copy(acc_v, out_hbm.at[tid])


_deg_call = functools.partial(
    pl.kernel,
    out_type=jax.ShapeDtypeStruct((NW, NP), _f32),
    mesh=_SC_MESH,
    compiler_params=_SC_PARAMS,
    scratch_types=[
        pltpu.VMEM((_DCH,), _i32),
        pltpu.VMEM((_DCH,), _f32),
        pltpu.VMEM((NP,), _f32),
    ],
)(_deg_body)


_NB = 2048


def _prep_body(xt_ref, degp_ref, wf_ref, wzh_ref, bf_ref, yp_ref, dis_ref):
    deg = jnp.sum(degp_ref[...], axis=0) + 1.0
    dis = lax.rsqrt(deg)
    dis_ref[...] = dis[None, :]
    wc = jnp.dot(wf_ref[...], wzh_ref[...], preferred_element_type=_f32)
    bc = jnp.dot(bf_ref[...], wzh_ref[...], preferred_element_type=_f32)
    for t in range(T):
        y = jnp.dot(xt_ref[t], wc, preferred_element_type=_f32) + bc
        o = (t % 2) * 64
        yp_ref[t // 2, :, o:o + 64] = y * dis[:, None]


def _prep_call(xt, degp, wf, wzh, bf):
    return pl.pallas_call(
        _prep_body,
        grid=(NP // _NB,),
        in_specs=[
            pl.BlockSpec((T, _NB, F), lambda i: (0, i, 0)),
            pl.BlockSpec((NW, _NB), lambda i: (0, i)),
            pl.BlockSpec((F, F), lambda i: (0, 0)),
            pl.BlockSpec((F, 2 * OUT), lambda i: (0, 0)),
            pl.BlockSpec((1, F), lambda i: (0, 0)),
        ],
        out_specs=[
            pl.BlockSpec((NCHUNK, _NB, CK), lambda i: (0, i, 0)),
            pl.BlockSpec((1, _NB), lambda i: (0, i)),
        ],
        out_shape=[
            jax.ShapeDtypeStruct((NCHUNK, NP, CK), _f32),
            jax.ShapeDtypeStruct((1, NP), _f32),
        ],
    )(xt, degp, wf, wzh, bf)


def _agg_body(ypf_hbm, row6_hbm, col3_hbm, ew3_hbm, out_hbm,
              colsb, rowsb, ewsb, gbuf0, gbuf1, sbuf0, sbuf1, a_sh,
              sem_g0, sem_g1, sem_s0, sem_s1):
    cc = lax.axis_index("c")
    ss = lax.axis_index("s")
    tid = ss * NC + cc
    gbufs = (gbuf0, gbuf1)
    gsems = (sem_g0, sem_g1)
    sbufs = (sbuf0, sbuf1)
    ssems = (sem_s0, sem_s1)

    def gzero():
        def zrow(i, _):
            for k in range(CK // L):
                gbuf0[i, pl.ds(k * L, L)] = jnp.zeros((L,), _f32)
            return 0

        lax.fori_loop(0, B, zrow, 0)

    def zero_slice():
        for j in range(SLICE // B):
            pltpu.sync_copy(gbuf0, a_sh.at[pl.ds(ss * SLICE + j * B, B)])

    def scale(kb, gbuf, sbuf):
        for e in range(B):
            w = plsc.load_gather(
                ewsb, [jnp.full((L,), kb, _i32), jnp.full((L,), e, _i32)])
            for k in range(CK // L):
                sl = pl.ds(k * L, L)
                sbuf[e, sl] = gbuf[e, sl] * w

    def wait_gather(kb, p):
        pltpu.make_async_copy(
            ypf_hbm.at[rowsb.at[kb]], gbufs[p], gsems[p]).wait()

    def issue_gather(kb, p):
        pltpu.async_copy(ypf_hbm.at[rowsb.at[kb]], gbufs[p], gsems[p])

    def issue_scatter(kb, p):
        pltpu.async_copy(sbufs[p], a_sh.at[colsb.at[kb]], ssems[p], add=True)

    def drain_scatter(p):
        pltpu.make_async_copy(sbufs[p], a_sh.at[colsb.at[0]], ssems[p]).wait()

    gzero()
    zero_slice()
    plsc.subcore_barrier()

    def chunk(c, _):
        def superb(s, _):
            pltpu.sync_copy(row6_hbm.at[c, tid, s], rowsb)
            pltpu.sync_copy(col3_hbm.at[tid, s], colsb)
            pltpu.sync_copy(ew3_hbm.at[tid, s], ewsb)
            issue_gather(0, 0)
            issue_gather(1, 1)
            for p in range(2):
                wait_gather(p, p)
                scale(p, gbufs[p], sbufs[p])
                issue_scatter(p, p)
                issue_gather(p + 2, p)

            def pair(jp, _):
                for p in range(2):
                    kb = 2 * jp + p
                    wait_gather(kb, p)
                    drain_scatter(p)
                    scale(kb, gbufs[p], sbufs[p])
                    issue_scatter(kb, p)
                    issue_gather(kb + 2, p)
                return 0

            lax.fori_loop(1, NPAIR - 1, pair, 0)
            for p in range(2):
                kb = SB - 2 + p
                wait_gather(kb, p)
                drain_scatter(p)
                scale(kb, gbufs[p], sbufs[p])
                issue_scatter(kb, p)
            drain_scatter(0)
            drain_scatter(1)
            return 0

        lax.fori_loop(0, NSB, superb, 0)
        plsc.subcore_barrier()
        pltpu.sync_copy(a_sh.at[pl.ds(ss * SLICE, SLICE)],
                        out_hbm.at[cc, c, pl.ds(ss * SLICE, SLICE)])
        gzero()
        zero_slice()
        plsc.subcore_barrier()
        return 0

    lax.fori_loop(0, NCHUNK, chunk, 0)


_agg_call = functools.partial(
    pl.kernel,
    out_type=jax.ShapeDtypeStruct((NC, NCHUNK, NP, CK), _f32),
    mesh=_SC_MESH,
    compiler_params=_SC_PARAMS,
    scratch_types=[
        pltpu.VMEM((SB, B), _i32),
        pltpu.VMEM((SB, B), _i32),
        pltpu.VMEM((SB, B), _f32),
        pltpu.VMEM((B, CK), _f32),
        pltpu.VMEM((B, CK), _f32),
        pltpu.VMEM((B, CK), _f32),
        pltpu.VMEM((B, CK), _f32),
        pltpu.VMEM_SHARED((NP, CK), _f32),
        pltpu.SemaphoreType.DMA,
        pltpu.SemaphoreType.DMA,
        pltpu.SemaphoreType.DMA,
        pltpu.SemaphoreType.DMA,
    ],
)(_agg_body)


_NB2 = 2048


def _fin_body(p_ref, yp_ref, dis_ref, att_ref, wlz_ref, blz_ref,
              wlh_ref, blh_ref, bz_ref, bh_ref, wo_ref, bo_ref, out_ref):
    att = att_ref[0]
    ex = jnp.exp(att - jnp.max(att))
    probs = ex / jnp.sum(ex)
    dis = dis_ref[0]
    h = jnp.zeros((_NB2, OUT), _f32)
    parr = p_ref[...]
    for c in range(NCHUNK):
        ac = (parr[0, c] + parr[1, c] + yp_ref[c]) * dis[:, None]
        for tt in range(2):
            t = 2 * c + tt
            o = tt * 64
            uz = ac[:, o:o + OUT] + bz_ref[...]
            uh = ac[:, o + OUT:o + 2 * OUT] + bh_ref[...]
            z = jax.nn.sigmoid(
                jnp.dot(uz, wlz_ref[...], preferred_element_type=_f32)
                + blz_ref[...])
            ht = jnp.tanh(
                jnp.dot(uh, wlh_ref[...], preferred_element_type=_f32)
                + blh_ref[...])
            h = h + probs[t] * (1.0 - z) * ht
    out_ref[...] = (
        jnp.dot(jnp.maximum(h, 0.0), wo_ref[...], preferred_element_type=_f32)
        + bo_ref[...])


def _final_call(p, yp, dis, att, wlz, blz, wlh, blh, bz, bh, wo, bo):
    small = lambda r, ccol: pl.BlockSpec((r, ccol), lambda i: (0, 0))
    return pl.pallas_call(
        _fin_body,
        grid=(NP // _NB2,),
        in_specs=[
            pl.BlockSpec((NC, NCHUNK, _NB2, CK), lambda i: (0, 0, i, 0)),
            pl.BlockSpec((NCHUNK, _NB2, CK), lambda i: (0, i, 0)),
            pl.BlockSpec((1, _NB2), lambda i: (0, i)),
            small(1, T),
            small(OUT, OUT), small(1, OUT),
            small(OUT, OUT), small(1, OUT),
            small(1, OUT), small(1, OUT),
            small(OUT, T), small(1, T),
        ],
        out_specs=pl.BlockSpec((_NB2, T), lambda i: (i, 0)),
        out_shape=jax.ShapeDtypeStruct((NP, T), _f32),
    )(p, yp, dis, att, wlz, blz, wlh, blh, bz, bh, wo, bo)


def kernel(x, edge_index, edge_features, W_feat, b_feat, W_edge, b_edge,
           att, Wz, bz, Wlz, blz, Wr, br, Wlr, blr, Wh, bh, Wlh, blh,
           W_out, b_out):
    row = edge_index[0]
    col = edge_index[1]
    ew, row6 = _ew_call(edge_features, row, W_edge, b_edge.reshape(1, 1))
    degp = _deg_call(col, ew)
    xt = jnp.pad(jnp.transpose(x, (2, 0, 1)), ((0, 0), (0, NP - N), (0, 0)))
    wzh = jnp.concatenate([Wz, Wh], axis=1)
    yp, dis = _prep_call(xt, degp, W_feat, wzh, b_feat.reshape(1, F))
    col3 = col.reshape(NW, NSB, SB, B)
    ew3 = ew.reshape(NW, NSB, SB, B)
    p = _agg_call(yp.reshape(NCHUNK * NP, CK), row6, col3, ew3)
    out = _final_call(
        p, yp, dis, att.reshape(1, T),
        Wlz[:OUT], blz.reshape(1, OUT),
        Wlh[:OUT], blh.reshape(1, OUT),
        bz.reshape(1, OUT), bh.reshape(1, OUT),
        W_out, b_out.reshape(1, T))
    return out[:N]
